# baseline (speedup 1.0000x reference)
import functools
import math

import jax
import jax.numpy as jnp
from jax import lax
from jax.experimental import pallas as pl
from jax.experimental.pallas import tpu as pltpu
from jax.experimental.pallas import tpu_sc as plsc

_EMBED_DIM = 64
_SCALE = math.sqrt(_EMBED_DIM)
_NC = 2
_NS = 16
_NW = _NC * _NS
_LANES = 16


def _sc_body(chunk, n_chunks, idx_hbm, base_hbm, tab_hbm, out_hbm,
             idx_v, xbuf, ebuf, sem_x, sem_e):
    b_per_w = chunk * n_chunks
    wid = lax.axis_index("s") * _NC + lax.axis_index("c")
    base = wid * b_per_w

    @pl.loop(0, n_chunks)
    def _chunk_loop(i):
        off = base + i * chunk
        pltpu.sync_copy(idx_hbm.at[pl.ds(off, chunk)], idx_v)
        cp_x = pltpu.async_copy(base_hbm.at[idx_v], xbuf, sem_x)
        cp_e = pltpu.async_copy(tab_hbm.at[idx_v], ebuf, sem_e)
        cp_x.wait()
        cp_e.wait()

        @plsc.parallel_loop(0, chunk, unroll=2)
        def _row_loop(r):
            for v in range(_EMBED_DIM // _LANES):
                sl = pl.ds(v * _LANES, _LANES)
                xbuf[r, sl] = xbuf[r, sl] + _SCALE * ebuf[r, sl]

        pltpu.sync_copy(xbuf, out_hbm.at[pl.ds(off, chunk)])


def _make_sc_kernel(n_tok, chunk):
    assert n_tok % (_NW * chunk) == 0
    n_chunks = n_tok // (_NW * chunk)
    mesh = plsc.VectorSubcoreMesh(core_axis_name="c", subcore_axis_name="s")
    return pl.kernel(
        functools.partial(_sc_body, chunk, n_chunks),
        out_type=jax.ShapeDtypeStruct((n_tok, _EMBED_DIM), jnp.float32),
        mesh=mesh,
        compiler_params=pltpu.CompilerParams(use_tc_tiling_on_sc=False),
        scratch_types=[
            pltpu.VMEM((chunk,), jnp.int32),
            pltpu.VMEM((chunk, _EMBED_DIM), jnp.float32),
            pltpu.VMEM((chunk, _EMBED_DIM), jnp.float32),
            pltpu.SemaphoreType.DMA,
            pltpu.SemaphoreType.DMA,
        ],
    )


def kernel(tokens, base_table, table):
    n_tok = tokens.shape[0] * tokens.shape[1]
    idx = jnp.asarray(tokens, jnp.int32).reshape(n_tok)
    out = _make_sc_kernel(n_tok, 800)(idx, base_table, table)
    return out.reshape(tokens.shape[0], tokens.shape[1], _EMBED_DIM)
